# Initial kernel scaffold; baseline (speedup 1.0000x reference)
import jax
import jax.numpy as jnp
from jax.experimental import pallas as pl


def kernel(logits, labels, logit_lengths, label_lengths):
    raise NotImplementedError("write your pallas kernel here")


# speedup vs baseline: 1.1761x; 1.1761x over previous
import jax
import jax.numpy as jnp
from jax.experimental import pallas as pl
from jax.experimental.pallas import tpu as pltpu

_TB = 16
_LANES = 128
_NEG = -1e30


def _shift_right(x, k, fill):
    pad = jnp.full(x.shape[:-1] + (k,), fill, x.dtype)
    return jnp.concatenate([pad, x[..., :-k]], axis=-1)


def _logaddexp(a, b):
    m = jnp.maximum(a, b)
    return m + jnp.log1p(jnp.exp(-jnp.abs(a - b)))


def _excl_cumsum(x):
    x = _shift_right(x, 1, 0.0)
    for k in (1, 2, 4, 8, 16, 32, 64):
        x = x + _shift_right(x, k, 0.0)
    return x


def _logcumsumexp(x):
    for k in (1, 2, 4, 8, 16, 32, 64):
        x = _logaddexp(x, _shift_right(x, k, _NEG))
    return x


def _rnnt_kernel(logits_ref, lab_ref, tl_ref, ul_ref, out_ref,
                 row_ref, bprev_ref):
    b = pl.program_id(0)
    tb = pl.program_id(1)
    x = logits_ref[0]
    n_t, up1, v = x.shape

    labm = lab_ref[0]
    vio = jax.lax.broadcasted_iota(jnp.int32, (up1, v), 1)
    maskf = jnp.where(vio == labm, 1.0, 0.0)
    mask0 = jnp.where(vio == 0, 1.0, 0.0)
    emit = jnp.sum(x * maskf[None], axis=-1)
    blank = jnp.sum(x * mask0[None], axis=-1)
    pad = jnp.zeros((n_t, _LANES - up1), jnp.float32)
    emit = jnp.concatenate([emit, pad], axis=-1)
    blank = jnp.concatenate([blank, pad], axis=-1)

    tl = tl_ref[b]
    ul = ul_ref[b]
    lane = jax.lax.broadcasted_iota(jnp.int32, (1, _LANES), 1)
    usel = jnp.where(lane == ul, 1.0, 0.0)

    row = row_ref[...]
    bprev = bprev_ref[...]
    vacc = jnp.zeros((1, _LANES), jnp.float32)
    for i in range(n_t):
        e_i = emit[i:i + 1, :]
        b_i = blank[i:i + 1, :]
        c_i = _excl_cumsum(e_i)
        a = row + bprev
        newrow = c_i + _logcumsumexp(a - c_i)
        if i == 0:
            newrow = jnp.where(tb == 0, c_i, newrow)
        t = tb * n_t + i
        vacc = vacc + jnp.where(t == tl, newrow * usel, 0.0)
        row = newrow
        bprev = b_i
    row_ref[...] = row
    bprev_ref[...] = bprev

    @pl.when(tb == 0)
    def _():
        out_ref[...] = jnp.zeros_like(out_ref)

    out_ref[...] = out_ref[...] + vacc[None]


def kernel(logits, labels, logit_lengths, label_lengths):
    B, T, up1, V = logits.shape
    U = up1 - 1
    tl = jnp.clip(logit_lengths, 1, T).astype(jnp.int32) - 1
    ul = jnp.clip(label_lengths, 1, U).astype(jnp.int32)
    labs = jnp.clip(labels, 0, V - 1).astype(jnp.int32)
    lab_col = jnp.concatenate(
        [labs, jnp.full((B, 1), -1, jnp.int32)], axis=1).reshape(B, up1, 1)

    out = pl.pallas_call(
        _rnnt_kernel,
        out_shape=jax.ShapeDtypeStruct((B, 1, _LANES), jnp.float32),
        grid=(B, T // _TB),
        in_specs=[
            pl.BlockSpec((1, _TB, up1, V), lambda b, t: (b, t, 0, 0)),
            pl.BlockSpec((1, up1, 1), lambda b, t: (b, 0, 0)),
            pl.BlockSpec(memory_space=pltpu.SMEM),
            pl.BlockSpec(memory_space=pltpu.SMEM),
        ],
        out_specs=pl.BlockSpec((1, 1, _LANES), lambda b, t: (b, 0, 0)),
        scratch_shapes=[
            pltpu.VMEM((1, _LANES), jnp.float32),
            pltpu.VMEM((1, _LANES), jnp.float32),
        ],
        compiler_params=pltpu.CompilerParams(
            dimension_semantics=("parallel", "arbitrary"),
        ),
        name="rnnt_alpha",
    )(logits, lab_col, tl, ul)
    return (-jnp.sum(out) / B).reshape(1)


# speedup vs baseline: 2.3232x; 1.9753x over previous
import functools

import jax
import jax.numpy as jnp
from jax.experimental import pallas as pl
from jax.experimental.pallas import tpu as pltpu

_TB = 16
_LANES = 128
_NEG = -1e30


def _logaddexp(a, b):
    m = jnp.maximum(a, b)
    return m + jnp.log1p(jnp.exp(-jnp.abs(a - b)))


def _reduce_kernel(logits_ref, lab_ref, emit_ref, blank_ref):
    x = logits_ref[0]
    n_t, up1, v = x.shape
    labm = lab_ref[0]
    vio = jax.lax.broadcasted_iota(jnp.int32, (up1, v), 1)
    maskf = jnp.where(vio == labm, 1.0, 0.0)
    mask0 = jnp.where(vio == 0, 1.0, 0.0)
    emit = jnp.sum(x * maskf[None], axis=-1)
    blank = jnp.sum(x * mask0[None], axis=-1)
    pad = jnp.zeros((n_t, _LANES - up1), jnp.float32)
    emit = jnp.concatenate([emit, pad], axis=-1)
    blank = jnp.concatenate([blank, pad], axis=-1)
    emit_ref[...] = emit[None]
    blank_ref[...] = blank[None]


def _skew(x, rows):
    t = x.shape[0]
    x = jnp.concatenate(
        [x, jnp.full((rows - t, _LANES), _NEG, jnp.float32)], axis=0)
    lane = jax.lax.broadcasted_iota(jnp.int32, (1, _LANES), 1)
    for k in (1, 2, 4, 8, 16, 32, 64):
        shifted = jnp.concatenate(
            [jnp.full((k, _LANES), _NEG, jnp.float32), x[:-k]], axis=0)
        x = jnp.where((lane & k) != 0, shifted, x)
    return x


def _alpha_kernel(emit_ref, blank_ref, tl_ref, ul_ref, out_ref,
                  se_ref, sb_ref, *, n_d):
    p = pl.program_id(0)
    rows = se_ref.shape[0]

    se0 = _skew(emit_ref[0], rows)
    se1 = _skew(emit_ref[1], rows)
    sb0 = _skew(blank_ref[0], rows)
    sb1 = _skew(blank_ref[1], rows)
    se_ref[...] = jnp.concatenate([se0[:, None], se1[:, None]], axis=1)
    sb_ref[...] = jnp.concatenate([sb0[:, None], sb1[:, None]], axis=1)

    lane = jax.lax.broadcasted_iota(jnp.int32, (2, _LANES), 1)
    sub = jax.lax.broadcasted_iota(jnp.int32, (2, _LANES), 0)
    tl0 = tl_ref[2 * p]
    tl1 = tl_ref[2 * p + 1]
    ul0 = ul_ref[2 * p]
    ul1 = ul_ref[2 * p + 1]
    dstar = jnp.where(sub == 0, tl0 + ul0, tl1 + ul1)
    usel = jnp.where(lane == jnp.where(sub == 0, ul0, ul1), 1.0, 0.0)

    d_init = jnp.where(lane == 0, 0.0, _NEG)
    vacc0 = jnp.zeros((2, _LANES), jnp.float32)
    negcol = jnp.full((2, 1), _NEG, jnp.float32)

    def body(d, carry):
        dvec, vacc = carry
        ed = se_ref[pl.ds(d - 1, 1), :, :][0]
        bd = sb_ref[pl.ds(d - 1, 1), :, :][0]
        tmp = dvec + ed
        sh = jnp.concatenate([negcol, tmp[:, :-1]], axis=-1)
        dnew = _logaddexp(dvec + bd, sh)
        vacc = vacc + jnp.where(dstar == d, dnew * usel, 0.0)
        return dnew, vacc

    _, vacc = jax.lax.fori_loop(1, n_d + 1, body, (d_init, vacc0))
    out_ref[0] = vacc


def kernel(logits, labels, logit_lengths, label_lengths):
    B, T, up1, V = logits.shape
    U = up1 - 1
    tl = jnp.clip(logit_lengths, 1, T).astype(jnp.int32) - 1
    ul = jnp.clip(label_lengths, 1, U).astype(jnp.int32)
    labs = jnp.clip(labels, 0, V - 1).astype(jnp.int32)
    lab_col = jnp.concatenate(
        [labs, jnp.full((B, 1), -1, jnp.int32)], axis=1).reshape(B, up1, 1)

    emit, blank = pl.pallas_call(
        _reduce_kernel,
        out_shape=(
            jax.ShapeDtypeStruct((B, T, _LANES), jnp.float32),
            jax.ShapeDtypeStruct((B, T, _LANES), jnp.float32),
        ),
        grid=(B, T // _TB),
        in_specs=[
            pl.BlockSpec((1, _TB, up1, V), lambda b, t: (b, t, 0, 0)),
            pl.BlockSpec((1, up1, 1), lambda b, t: (b, 0, 0)),
        ],
        out_specs=(
            pl.BlockSpec((1, _TB, _LANES), lambda b, t: (b, t, 0)),
            pl.BlockSpec((1, _TB, _LANES), lambda b, t: (b, t, 0)),
        ),
        compiler_params=pltpu.CompilerParams(
            dimension_semantics=("parallel", "parallel"),
        ),
        name="rnnt_reduce",
    )(logits, lab_col)

    out = pl.pallas_call(
        functools.partial(_alpha_kernel, n_d=T - 1 + U),
        out_shape=jax.ShapeDtypeStruct((B // 2, 2, _LANES), jnp.float32),
        grid=(B // 2,),
        in_specs=[
            pl.BlockSpec((2, T, _LANES), lambda p: (p, 0, 0)),
            pl.BlockSpec((2, T, _LANES), lambda p: (p, 0, 0)),
            pl.BlockSpec(memory_space=pltpu.SMEM),
            pl.BlockSpec(memory_space=pltpu.SMEM),
        ],
        out_specs=pl.BlockSpec((1, 2, _LANES), lambda p: (p, 0, 0)),
        scratch_shapes=[
            pltpu.VMEM((T + _LANES, 2, _LANES), jnp.float32),
            pltpu.VMEM((T + _LANES, 2, _LANES), jnp.float32),
        ],
        compiler_params=pltpu.CompilerParams(
            dimension_semantics=("parallel",),
        ),
        name="rnnt_alpha",
    )(emit, blank, tl, ul)
    return (-jnp.sum(out) / B).reshape(1)


# speedup vs baseline: 2.3698x; 1.0201x over previous
import functools

import jax
import jax.numpy as jnp
from jax.experimental import pallas as pl
from jax.experimental.pallas import tpu as pltpu

_TB = 16
_LANES = 128
_NEG = -1e30


def _logaddexp(a, b):
    m = jnp.maximum(a, b)
    return m + jnp.log1p(jnp.exp(-jnp.abs(a - b)))


def _reduce_kernel(logits_ref, lab_ref, emit_ref, blank_ref):
    x = logits_ref[0]
    n_t, up1, v = x.shape
    labm = lab_ref[0]
    vio = jax.lax.broadcasted_iota(jnp.int32, (up1, v), 1)
    maskf = jnp.where(vio == labm, 1.0, 0.0)
    mask0 = jnp.where(vio == 0, 1.0, 0.0)
    emit = jnp.sum(x * maskf[None], axis=-1)
    blank = jnp.sum(x * mask0[None], axis=-1)
    pad = jnp.zeros((n_t, _LANES - up1), jnp.float32)
    emit = jnp.concatenate([emit, pad], axis=-1)
    blank = jnp.concatenate([blank, pad], axis=-1)
    emit_ref[...] = emit[None]
    blank_ref[...] = blank[None]


def _skew(x, rows):
    t = x.shape[0]
    x = jnp.concatenate(
        [x, jnp.full((rows - t, _LANES), _NEG, jnp.float32)], axis=0)
    lane = jax.lax.broadcasted_iota(jnp.int32, (1, _LANES), 1)
    for k in (1, 2, 4, 8, 16, 32, 64):
        shifted = jnp.concatenate(
            [jnp.full((k, _LANES), _NEG, jnp.float32), x[:-k]], axis=0)
        x = jnp.where((lane & k) != 0, shifted, x)
    return x


def _alpha_kernel(emit_ref, blank_ref, tl_ref, ul_ref, out_ref,
                  se_ref, sb_ref, *, n_d):
    p = pl.program_id(0)
    rows = se_ref.shape[0]

    se0 = _skew(emit_ref[0], rows)
    se1 = _skew(emit_ref[1], rows)
    sb0 = _skew(blank_ref[0], rows)
    sb1 = _skew(blank_ref[1], rows)
    se_ref[...] = jnp.concatenate([se0[:, None], se1[:, None]], axis=1)
    sb_ref[...] = jnp.concatenate([sb0[:, None], sb1[:, None]], axis=1)

    lane = jax.lax.broadcasted_iota(jnp.int32, (2, _LANES), 1)
    sub = jax.lax.broadcasted_iota(jnp.int32, (2, _LANES), 0)
    tl0 = tl_ref[2 * p]
    tl1 = tl_ref[2 * p + 1]
    ul0 = ul_ref[2 * p]
    ul1 = ul_ref[2 * p + 1]
    dstar = jnp.where(sub == 0, tl0 + ul0, tl1 + ul1)
    usel_b = lane == jnp.where(sub == 0, ul0, ul1)

    d_init = jnp.where(lane == 0, 0.0, _NEG)
    vacc0 = jnp.zeros((2, _LANES), jnp.float32)
    negcol = jnp.full((2, 1), _NEG, jnp.float32)

    def body(d, carry):
        dvec, vacc = carry
        ed = se_ref[pl.ds(d - 1, 1), :, :][0]
        bd = sb_ref[pl.ds(d - 1, 1), :, :][0]
        tmp = dvec + ed
        sh = jnp.concatenate([negcol, tmp[:, :-1]], axis=-1)
        dnew = _logaddexp(dvec + bd, sh)
        vacc = vacc + jnp.where(dstar == d, jnp.where(usel_b, dnew, 0.0), 0.0)
        return dnew, vacc

    d_hi = jnp.minimum(jnp.maximum(tl0 + ul0, tl1 + ul1), n_d)
    _, vacc = jax.lax.fori_loop(1, d_hi + 1, body, (d_init, vacc0))
    out_ref[0] = vacc


def kernel(logits, labels, logit_lengths, label_lengths):
    B, T, up1, V = logits.shape
    U = up1 - 1
    tl = jnp.clip(logit_lengths, 1, T).astype(jnp.int32) - 1
    ul = jnp.clip(label_lengths, 1, U).astype(jnp.int32)
    labs = jnp.clip(labels, 0, V - 1).astype(jnp.int32)
    lab_col = jnp.concatenate(
        [labs, jnp.full((B, 1), -1, jnp.int32)], axis=1).reshape(B, up1, 1)

    emit, blank = pl.pallas_call(
        _reduce_kernel,
        out_shape=(
            jax.ShapeDtypeStruct((B, T, _LANES), jnp.float32),
            jax.ShapeDtypeStruct((B, T, _LANES), jnp.float32),
        ),
        grid=(2, B // 2, T // _TB),
        in_specs=[
            pl.BlockSpec((1, _TB, up1, V), lambda c, i, t: (c * 2 + i, t, 0, 0)),
            pl.BlockSpec((1, up1, 1), lambda c, i, t: (c * 2 + i, 0, 0)),
        ],
        out_specs=(
            pl.BlockSpec((1, _TB, _LANES), lambda c, i, t: (c * 2 + i, t, 0)),
            pl.BlockSpec((1, _TB, _LANES), lambda c, i, t: (c * 2 + i, t, 0)),
        ),
        compiler_params=pltpu.CompilerParams(
            dimension_semantics=("parallel", "parallel", "parallel"),
        ),
        name="rnnt_reduce",
    )(logits, lab_col)

    out = pl.pallas_call(
        functools.partial(_alpha_kernel, n_d=T - 1 + U),
        out_shape=jax.ShapeDtypeStruct((B // 2, 2, _LANES), jnp.float32),
        grid=(B // 2,),
        in_specs=[
            pl.BlockSpec((2, T, _LANES), lambda p: (p, 0, 0)),
            pl.BlockSpec((2, T, _LANES), lambda p: (p, 0, 0)),
            pl.BlockSpec(memory_space=pltpu.SMEM),
            pl.BlockSpec(memory_space=pltpu.SMEM),
        ],
        out_specs=pl.BlockSpec((1, 2, _LANES), lambda p: (p, 0, 0)),
        scratch_shapes=[
            pltpu.VMEM((T + _LANES, 2, _LANES), jnp.float32),
            pltpu.VMEM((T + _LANES, 2, _LANES), jnp.float32),
        ],
        compiler_params=pltpu.CompilerParams(
            dimension_semantics=("parallel",),
        ),
        name="rnnt_alpha",
    )(emit, blank, tl, ul)
    return (-jnp.sum(out) / B).reshape(1)
